# baseline (speedup 1.0000x reference)
import functools

import jax
import jax.numpy as jnp
from jax import lax
from jax.experimental import pallas as pl
from jax.experimental.pallas import tpu as pltpu
from jax.experimental.pallas import tpu_sc as plsc

_ED = 64
_LD = 64
_CHUNK = 128
_NC = 2
_NS = 16
_NW = _NC * _NS


def _gather_body(nch, table_hbm, idx_hbm, out_hbm, idx_v, rows_v, sem0, sem1):
  wid = lax.axis_index("s") * _NC + lax.axis_index("c")
  base = wid * (nch * _CHUNK)
  pltpu.sync_copy(idx_hbm.at[wid], idx_v)

  sems = (sem0, sem1)

  def start(c, buf):
    pltpu.make_async_copy(
        table_hbm.at[idx_v.at[c]], rows_v.at[buf], sems[buf]).start()

  def wait(c, buf):
    pltpu.make_async_copy(
        table_hbm.at[idx_v.at[c]], rows_v.at[buf], sems[buf]).wait()

  def store(c, buf):
    pltpu.sync_copy(rows_v.at[buf], out_hbm.at[pl.ds(base + c * _CHUNK, _CHUNK)])

  start(0, 0)

  def body(g, carry):
    c = 2 * g
    start(c + 1, 1)
    wait(c, 0)
    store(c, 0)

    @pl.when(c + 2 < nch)
    def _():
      start(c + 2, 0)

    wait(c + 1, 1)
    store(c + 1, 1)
    return carry

  lax.fori_loop(0, nch // 2, body, 0)


def _sc_gather(table, ids3):
  nw, nch, chunk = ids3.shape
  n = nw * nch * chunk
  mesh = plsc.VectorSubcoreMesh(core_axis_name="c", subcore_axis_name="s")
  f = functools.partial(
      pl.kernel,
      mesh=mesh,
      out_type=jax.ShapeDtypeStruct((n, _ED), jnp.float32),
      scratch_types=[
          pltpu.VMEM((nch, chunk), jnp.int32),
          pltpu.VMEM((2, chunk, _ED), jnp.float32),
          pltpu.SemaphoreType.DMA,
          pltpu.SemaphoreType.DMA,
      ],
      compiler_params=pltpu.CompilerParams(use_tc_tiling_on_sc=False),
  )(functools.partial(_gather_body, nch))
  return f(table, ids3)


def _proj_body(x_ref, w_ref, b_ref, pos_ref, o_ref):
  bb, ll, ed = x_ref.shape
  x = x_ref[...].reshape(bb * ll, ed)
  y = jnp.dot(x, w_ref[...], preferred_element_type=jnp.float32)
  o_ref[...] = y.reshape(bb, ll, -1) + b_ref[...] + pos_ref[...]


def _tc_project(x, proj_w, proj_b, pos_embed):
  b, l, ed = x.shape
  ld = proj_w.shape[1]
  bb = 64
  grid = (b // bb,)
  return pl.pallas_call(
      _proj_body,
      grid=grid,
      in_specs=[
          pl.BlockSpec((bb, l, ed), lambda i: (i, 0, 0)),
          pl.BlockSpec((ed, ld), lambda i: (0, 0)),
          pl.BlockSpec((1, 1, ld), lambda i: (0, 0, 0)),
          pl.BlockSpec((1, l, ld), lambda i: (0, 0, 0)),
      ],
      out_specs=pl.BlockSpec((bb, l, ld), lambda i: (i, 0, 0)),
      out_shape=jax.ShapeDtypeStruct((b, l, ld), jnp.float32),
  )(x, proj_w, proj_b.reshape(1, 1, ld), pos_embed)


def kernel(token_ids, embed_table, proj_w, proj_b, pos_embed):
  b, l = token_ids.shape
  n = b * l
  assert n % (_NW * _CHUNK) == 0
  nch = n // (_NW * _CHUNK)
  ids3 = token_ids.astype(jnp.int32).reshape(_NW, nch, _CHUNK)
  rows = _sc_gather(embed_table, ids3)
  x = rows.reshape(b, l, _ED)
  return _tc_project(x, proj_w, proj_b, pos_embed)


# speedup vs baseline: 1.0513x; 1.0513x over previous
import functools

import jax
import jax.numpy as jnp
from jax import lax
from jax.experimental import pallas as pl
from jax.experimental.pallas import tpu as pltpu
from jax.experimental.pallas import tpu_sc as plsc

_ED = 64
_LD = 64
_GD = 2 * _ED
_CHUNK = 128
_NC = 2
_NS = 16
_NW = _NC * _NS


def _gather_body(nch, table_hbm, idx_hbm, out_hbm, idx_v, rows_v, sem0, sem1):
  wid = lax.axis_index("s") * _NC + lax.axis_index("c")
  base = wid * (nch * _CHUNK)
  pltpu.sync_copy(idx_hbm.at[wid], idx_v)

  sems = (sem0, sem1)

  def start(c, buf):
    pltpu.make_async_copy(
        table_hbm.at[idx_v.at[c]], rows_v.at[buf], sems[buf]).start()

  def wait(c, buf):
    pltpu.make_async_copy(
        table_hbm.at[idx_v.at[c]], rows_v.at[buf], sems[buf]).wait()

  def store(c, buf):
    pltpu.sync_copy(rows_v.at[buf], out_hbm.at[pl.ds(base + c * _CHUNK, _CHUNK)])

  start(0, 0)

  def body(g, carry):
    c = 2 * g
    start(c + 1, 1)
    wait(c, 0)
    store(c, 0)

    @pl.when(c + 2 < nch)
    def _():
      start(c + 2, 0)

    wait(c + 1, 1)
    store(c + 1, 1)
    return carry

  lax.fori_loop(0, nch // 2, body, 0)


def _sc_gather(table2, ids3):
  nw, nch, chunk = ids3.shape
  n = nw * nch * chunk
  mesh = plsc.VectorSubcoreMesh(core_axis_name="c", subcore_axis_name="s")
  f = functools.partial(
      pl.kernel,
      mesh=mesh,
      out_type=jax.ShapeDtypeStruct((n, _GD), jnp.float32),
      scratch_types=[
          pltpu.VMEM((nch, chunk), jnp.int32),
          pltpu.VMEM((2, chunk, _GD), jnp.float32),
          pltpu.SemaphoreType.DMA,
          pltpu.SemaphoreType.DMA,
      ],
  )(functools.partial(_gather_body, nch))
  return f(table2, ids3)


def _proj_body(x_ref, ids_ref, w2_ref, b_ref, pos_ref, o_ref):
  bb, ll, gd = x_ref.shape
  x = x_ref[...]
  par = (ids_ref[...] & 1)[:, :, None]
  lane = lax.broadcasted_iota(jnp.int32, (1, 1, gd), 2)
  m = ((lane >= _ED) == (par == 1)).astype(jnp.float32)
  xm = (x * m).reshape(bb * ll, gd)
  y = jnp.dot(xm, w2_ref[...], preferred_element_type=jnp.float32)
  o_ref[...] = y.reshape(bb, ll, -1) + b_ref[...] + pos_ref[...]


def _tc_project(x, token_ids, w2, proj_b, pos_embed):
  b, l, gd = x.shape
  ld = w2.shape[1]
  bb = 64
  grid = (b // bb,)
  return pl.pallas_call(
      _proj_body,
      grid=grid,
      in_specs=[
          pl.BlockSpec((bb, l, gd), lambda i: (i, 0, 0)),
          pl.BlockSpec((bb, l), lambda i: (i, 0)),
          pl.BlockSpec((gd, ld), lambda i: (0, 0)),
          pl.BlockSpec((1, 1, ld), lambda i: (0, 0, 0)),
          pl.BlockSpec((1, l, ld), lambda i: (0, 0, 0)),
      ],
      out_specs=pl.BlockSpec((bb, l, ld), lambda i: (i, 0, 0)),
      out_shape=jax.ShapeDtypeStruct((b, l, ld), jnp.float32),
  )(x, token_ids, w2, proj_b.reshape(1, 1, ld), pos_embed)


def kernel(token_ids, embed_table, proj_w, proj_b, pos_embed):
  b, l = token_ids.shape
  v, ed = embed_table.shape
  n = b * l
  assert n % (_NW * _CHUNK) == 0
  nch = n // (_NW * _CHUNK)
  ids = token_ids.astype(jnp.int32)
  table2 = embed_table.reshape(v // 2, _GD)
  phys3 = (ids >> 1).reshape(_NW, nch, _CHUNK)
  rows = _sc_gather(table2, phys3)
  x = rows.reshape(b, l, _GD)
  w2 = jnp.concatenate([proj_w, proj_w], axis=0)
  return _tc_project(x, ids, w2, proj_b, pos_embed)


# speedup vs baseline: 1.7099x; 1.6265x over previous
import functools

import jax
import jax.numpy as jnp
from jax import lax
from jax.experimental import pallas as pl
from jax.experimental.pallas import tpu as pltpu
from jax.experimental.pallas import tpu_sc as plsc

_ED = 64
_LD = 64
_GD = 2 * _ED
_S = 8192
_NB = 62
_D = _NB * _S
_V = 1000000
_CHUNK = 128
_NC = 2
_NS = 16
_NW = _NC * _NS


def _project_body(alo_ref, ahi_ref, w_ref, b_ref, o_ref):
  i = pl.program_id(0)
  w = w_ref[...]
  dn = (((0,), (0,)), ((), ()))
  ylo = lax.dot_general(alo_ref[...], w, dn, preferred_element_type=jnp.float32)
  yhi = lax.dot_general(ahi_ref[...], w, dn, preferred_element_type=jnp.float32)
  row = lax.broadcasted_iota(jnp.int32, (_S, _LD), 0) + i * _S
  yhi = jnp.where(row < _V - _D, yhi, 0.0)
  b = b_ref[...]
  o_ref[...] = jnp.concatenate([ylo + b, yhi + b], axis=1)


def _project(table_t, proj_w, proj_b):
  return pl.pallas_call(
      _project_body,
      grid=(_NB,),
      in_specs=[
          pl.BlockSpec((_ED, _S), lambda i: (0, i)),
          pl.BlockSpec((_ED, _S), lambda i: (0, jnp.minimum(i + _NB, _V // _S))),
          pl.BlockSpec((_ED, _LD), lambda i: (0, 0)),
          pl.BlockSpec((1, _LD), lambda i: (0, 0)),
      ],
      out_specs=pl.BlockSpec((_S, _GD), lambda i: (i, 0)),
      out_shape=jax.ShapeDtypeStruct((_D, _GD), jnp.float32),
  )(table_t, table_t, proj_w, proj_b.reshape(1, _LD))


def _gather_body(nch, table_hbm, idx_hbm, out_hbm, idx_v, rows_v, sem0, sem1):
  wid = lax.axis_index("s") * _NC + lax.axis_index("c")
  base = wid * (nch * _CHUNK)
  pltpu.sync_copy(idx_hbm.at[wid], idx_v)

  sems = (sem0, sem1)

  def start(c, buf):
    pltpu.make_async_copy(
        table_hbm.at[idx_v.at[c]], rows_v.at[buf], sems[buf]).start()

  def wait(c, buf):
    pltpu.make_async_copy(
        table_hbm.at[idx_v.at[c]], rows_v.at[buf], sems[buf]).wait()

  def store(c, buf):
    pltpu.sync_copy(rows_v.at[buf], out_hbm.at[pl.ds(base + c * _CHUNK, _CHUNK)])

  start(0, 0)

  def body(g, carry):
    c = 2 * g
    start(c + 1, 1)
    wait(c, 0)
    store(c, 0)

    @pl.when(c + 2 < nch)
    def _():
      start(c + 2, 0)

    wait(c + 1, 1)
    store(c + 1, 1)
    return carry

  lax.fori_loop(0, nch // 2, body, 0)


def _sc_gather(p2, ids3):
  nw, nch, chunk = ids3.shape
  n = nw * nch * chunk
  mesh = plsc.VectorSubcoreMesh(core_axis_name="c", subcore_axis_name="s")
  f = functools.partial(
      pl.kernel,
      mesh=mesh,
      out_type=jax.ShapeDtypeStruct((n, _GD), jnp.float32),
      scratch_types=[
          pltpu.VMEM((nch, chunk), jnp.int32),
          pltpu.VMEM((2, chunk, _GD), jnp.float32),
          pltpu.SemaphoreType.DMA,
          pltpu.SemaphoreType.DMA,
      ],
  )(functools.partial(_gather_body, nch))
  return f(p2, ids3)


_BB = 128
_LL = 40


def _finish_body(g_ref, ids_ref, pos_ref, o_ref):
  g = g_ref[...]
  eye = (lax.broadcasted_iota(jnp.int32, (_BB, _BB), 0) ==
         lax.broadcasted_iota(jnp.int32, (_BB, _BB), 1)).astype(jnp.float32)
  dn = (((0,), (0,)), ((), ()))
  yt = lax.dot_general(g, eye, dn, preferred_element_type=jnp.float32,
                       precision=lax.Precision.HIGHEST)
  par = (ids_ref[...] >= _D)[:, None, :]
  par = jnp.broadcast_to(par, (_LL, _GD, _BB)).reshape(_LL * _GD, _BB)
  j = lax.broadcasted_iota(jnp.int32, (_LL * _GD, _BB), 0) % _GD
  ym = jnp.where((j >= _ED) == par, yt, 0.0)
  y3 = ym.reshape(_LL, _GD, _BB)
  yv = y3[:, :_ED, :] + y3[:, _ED:, :]
  o_ref[...] = yv + pos_ref[...]


def _finish(g2, ids_t, pos3, b, l):
  grid = (l // _LL, b // _BB)
  return pl.pallas_call(
      _finish_body,
      grid=grid,
      in_specs=[
          pl.BlockSpec((_BB, _LL * _GD), lambda il, ib: (ib, il)),
          pl.BlockSpec((_LL, _BB), lambda il, ib: (il, ib)),
          pl.BlockSpec((_LL, _LD, 1), lambda il, ib: (il, 0, 0)),
      ],
      out_specs=pl.BlockSpec((_LL, _LD, _BB), lambda il, ib: (il, 0, ib)),
      out_shape=jax.ShapeDtypeStruct((l, _LD, b), jnp.float32),
  )(g2, ids_t, pos3)


def kernel(token_ids, embed_table, proj_w, proj_b, pos_embed):
  b, l = token_ids.shape
  n = b * l
  assert n % (_NW * _CHUNK) == 0
  nch = n // (_NW * _CHUNK)
  ids = token_ids.astype(jnp.int32)
  table_t = embed_table.T
  p2 = _project(table_t, proj_w, proj_b)
  phys = jnp.where(ids < _D, ids, ids - _D)
  ids3 = phys.reshape(_NW, nch, _CHUNK)
  g = _sc_gather(p2, ids3)
  g2 = g.reshape(b, l * _GD)
  ids_t = ids.T
  pos3 = pos_embed[0, :l, :][:, :, None]
  out_t = _finish(g2, ids_t, pos3, b, l)
  return jnp.transpose(out_t, (2, 0, 1))


# speedup vs baseline: 1.9151x; 1.1200x over previous
import functools

import jax
import jax.numpy as jnp
from jax import lax
from jax.experimental import pallas as pl
from jax.experimental.pallas import tpu as pltpu
from jax.experimental.pallas import tpu_sc as plsc

_ED = 64
_LD = 64
_GD = 2 * _ED
_S = 8192
_NB = 62
_D = _NB * _S
_V = 1000000
_CHUNK = 128
_NC = 2
_NS = 16
_NW = _NC * _NS


def _project_body(alo_ref, ahi_ref, wd_ref, o_ref):
  i = pl.program_id(0)
  lane_g = lax.broadcasted_iota(jnp.int32, (_ED, _S), 1) + i * _S
  ahi = jnp.where(lane_g >= _V - _D, 0.0, ahi_ref[...])
  a2 = jnp.concatenate([alo_ref[...], ahi], axis=0)
  dn = (((0,), (0,)), ((), ()))
  o_ref[...] = lax.dot_general(a2, wd_ref[...], dn,
                               preferred_element_type=jnp.float32)


def _project(table_t, wd):
  return pl.pallas_call(
      _project_body,
      grid=(_NB,),
      in_specs=[
          pl.BlockSpec((_ED, _S), lambda i: (0, i)),
          pl.BlockSpec((_ED, _S), lambda i: (0, jnp.minimum(i + _NB, _V // _S))),
          pl.BlockSpec((_GD, _GD), lambda i: (0, 0)),
      ],
      out_specs=pl.BlockSpec((_S, _GD), lambda i: (i, 0)),
      out_shape=jax.ShapeDtypeStruct((_D, _GD), jnp.float32),
  )(table_t, table_t, wd)


def _gather_body(nch, table_hbm, idx_hbm, out_hbm, idx_v, rows_v, sem0, sem1):
  wid = lax.axis_index("s") * _NC + lax.axis_index("c")
  base = wid * (nch * _CHUNK)
  pltpu.sync_copy(idx_hbm.at[wid], idx_v)

  sems = (sem0, sem1)

  def start(c, buf):
    pltpu.make_async_copy(
        table_hbm.at[idx_v.at[c]], rows_v.at[buf], sems[buf]).start()

  def wait(c, buf):
    pltpu.make_async_copy(
        table_hbm.at[idx_v.at[c]], rows_v.at[buf], sems[buf]).wait()

  def store(c, buf):
    pltpu.sync_copy(rows_v.at[buf], out_hbm.at[pl.ds(base + c * _CHUNK, _CHUNK)])

  start(0, 0)

  def body(g, carry):
    c = 2 * g
    start(c + 1, 1)
    wait(c, 0)
    store(c, 0)

    @pl.when(c + 2 < nch)
    def _():
      start(c + 2, 0)

    wait(c + 1, 1)
    store(c + 1, 1)
    return carry

  lax.fori_loop(0, nch // 2, body, 0)


def _sc_gather(p2, ids3):
  nw, nch, chunk = ids3.shape
  n = nw * nch * chunk
  mesh = plsc.VectorSubcoreMesh(core_axis_name="c", subcore_axis_name="s")
  f = functools.partial(
      pl.kernel,
      mesh=mesh,
      out_type=jax.ShapeDtypeStruct((n, _GD), jnp.float32),
      scratch_types=[
          pltpu.VMEM((nch, chunk), jnp.int32),
          pltpu.VMEM((2, chunk, _GD), jnp.float32),
          pltpu.SemaphoreType.DMA,
          pltpu.SemaphoreType.DMA,
      ],
  )(functools.partial(_gather_body, nch))
  return f(p2, ids3)


_BB = 128
_LL = 40


def _finish_body(g_ref, ids_ref, pos_ref, o_ref):
  g = g_ref[...]
  eye = (lax.broadcasted_iota(jnp.int32, (_BB, _BB), 0) ==
         lax.broadcasted_iota(jnp.int32, (_BB, _BB), 1)).astype(jnp.float32)
  dn = (((0,), (0,)), ((), ()))
  yt = lax.dot_general(g, eye, dn, preferred_element_type=jnp.float32,
                       precision=lax.Precision.HIGHEST)
  par = (ids_ref[...] >= _D)[:, None, :]
  par = jnp.broadcast_to(par, (_LL, _GD, _BB)).reshape(_LL * _GD, _BB)
  j = lax.broadcasted_iota(jnp.int32, (_LL * _GD, _BB), 0) % _GD
  ym = jnp.where((j >= _ED) == par, yt, 0.0)
  y3 = ym.reshape(_LL, _GD, _BB)
  yv = y3[:, :_ED, :] + y3[:, _ED:, :]
  o_ref[...] = yv + pos_ref[...]


def _finish(g2, ids_t, pos3, b, l):
  grid = (l // _LL, b // _BB)
  return pl.pallas_call(
      _finish_body,
      grid=grid,
      in_specs=[
          pl.BlockSpec((_BB, _LL * _GD), lambda il, ib: (ib, il)),
          pl.BlockSpec((_LL, _BB), lambda il, ib: (il, ib)),
          pl.BlockSpec((_LL, _LD, 1), lambda il, ib: (il, 0, 0)),
      ],
      out_specs=pl.BlockSpec((_LL, _LD, _BB), lambda il, ib: (il, 0, ib)),
      out_shape=jax.ShapeDtypeStruct((l, _LD, b), jnp.float32),
  )(g2, ids_t, pos3)


def kernel(token_ids, embed_table, proj_w, proj_b, pos_embed):
  b, l = token_ids.shape
  n = b * l
  assert n % (_NW * _CHUNK) == 0
  nch = n // (_NW * _CHUNK)
  ids = token_ids.astype(jnp.int32)
  table_t = embed_table.T
  z = jnp.zeros((_ED, _ED), jnp.float32)
  wd = jnp.concatenate([jnp.concatenate([proj_w, z], 1),
                        jnp.concatenate([z, proj_w], 1)], 0)
  p2 = _project(table_t, wd)
  phys = jnp.where(ids < _D, ids, ids - _D)
  ids3 = phys.reshape(_NW, nch, _CHUNK)
  g = _sc_gather(p2, ids3)
  g2 = g.reshape(b, l * _GD)
  ids_t = ids.T
  pos3 = (pos_embed[0, :l, :] + proj_b[None, :])[:, :, None]
  out_t = _finish(g2, ids_t, pos3, b, l)
  return jnp.transpose(out_t, (2, 0, 1))


# speedup vs baseline: 2.1524x; 1.1239x over previous
import functools

import jax
import jax.numpy as jnp
from jax import lax
from jax.experimental import pallas as pl
from jax.experimental.pallas import tpu as pltpu
from jax.experimental.pallas import tpu_sc as plsc

_ED = 64
_LD = 64
_GD = 2 * _ED
_S = 8192
_NB = 62
_D = _NB * _S
_V = 1000000
_CHUNK = 128
_NC = 2
_NS = 16
_NW = _NC * _NS


def _project_body(alo_ref, ahi_ref, wd_ref, o_ref):
  i = pl.program_id(0)
  lane_g = lax.broadcasted_iota(jnp.int32, (_ED, _S), 1) + i * _S
  ahi = jnp.where(lane_g >= _V - _D, 0.0, ahi_ref[...])
  a2 = jnp.concatenate([alo_ref[...], ahi], axis=0)
  dn = (((0,), (0,)), ((), ()))
  o_ref[...] = lax.dot_general(a2, wd_ref[...], dn,
                               preferred_element_type=jnp.float32)


def _project(table_t, wd):
  return pl.pallas_call(
      _project_body,
      grid=(_NB,),
      in_specs=[
          pl.BlockSpec((_ED, _S), lambda i: (0, i)),
          pl.BlockSpec((_ED, _S), lambda i: (0, jnp.minimum(i + _NB, _V // _S))),
          pl.BlockSpec((_GD, _GD), lambda i: (0, 0)),
      ],
      out_specs=pl.BlockSpec((_S, _GD), lambda i: (i, 0)),
      out_shape=jax.ShapeDtypeStruct((_D, _GD), jnp.float32),
  )(table_t, table_t, wd)


def _gather_body(nch, table_hbm, idx_hbm, out_hbm, idx_v, rows_v, sem0, sem1):
  wid = lax.axis_index("s") * _NC + lax.axis_index("c")
  base = wid * (nch * _CHUNK)
  pltpu.sync_copy(idx_hbm.at[wid], idx_v)

  sems = (sem0, sem1)

  def start(c, buf):
    pltpu.make_async_copy(
        table_hbm.at[idx_v.at[c]], rows_v.at[buf], sems[buf]).start()

  def wait(c, buf):
    pltpu.make_async_copy(
        table_hbm.at[idx_v.at[c]], rows_v.at[buf], sems[buf]).wait()

  def store(c, buf):
    pltpu.sync_copy(rows_v.at[buf], out_hbm.at[pl.ds(base + c * _CHUNK, _CHUNK)])

  start(0, 0)

  def body(g, carry):
    c = 2 * g
    start(c + 1, 1)
    wait(c, 0)
    store(c, 0)

    @pl.when(c + 2 < nch)
    def _():
      start(c + 2, 0)

    wait(c + 1, 1)
    store(c + 1, 1)
    return carry

  lax.fori_loop(0, nch // 2, body, 0)


def _sc_gather(p2, ids3):
  nw, nch, chunk = ids3.shape
  n = nw * nch * chunk
  mesh = plsc.VectorSubcoreMesh(core_axis_name="c", subcore_axis_name="s")
  f = functools.partial(
      pl.kernel,
      mesh=mesh,
      out_type=jax.ShapeDtypeStruct((n, _GD), jnp.float32),
      scratch_types=[
          pltpu.VMEM((nch, chunk), jnp.int32),
          pltpu.VMEM((2, chunk, _GD), jnp.float32),
          pltpu.SemaphoreType.DMA,
          pltpu.SemaphoreType.DMA,
      ],
  )(functools.partial(_gather_body, nch))
  return f(p2, ids3)


_BB = 128
_LL = 40


def _finish_body(g_ref, ids_ref, pos_ref, o_ref):
  eye = (lax.broadcasted_iota(jnp.int32, (_BB, _BB), 0) ==
         lax.broadcasted_iota(jnp.int32, (_BB, _BB), 1)).astype(jnp.float32)
  dn = (((0,), (0,)), ((), ()))
  yts = []
  for li in range(_LL):
    gl = g_ref[:, li, :]
    yts.append(lax.dot_general(gl, eye, dn, preferred_element_type=jnp.float32,
                               precision=lax.Precision.HIGHEST))
  yt = jnp.stack(yts, axis=0)
  par = (ids_ref[...] >= _D)[:, None, :]
  j = lax.broadcasted_iota(jnp.int32, (_LL, _GD, _BB), 1)
  ym = jnp.where((j >= _ED) == par, yt, 0.0)
  yv = ym[:, :_ED, :] + ym[:, _ED:, :]
  o_ref[...] = yv + pos_ref[...]


def _finish(g3, ids_t, pos3, b, l):
  grid = (l // _LL, b // _BB)
  return pl.pallas_call(
      _finish_body,
      grid=grid,
      in_specs=[
          pl.BlockSpec((_BB, _LL, _GD), lambda il, ib: (ib, il, 0)),
          pl.BlockSpec((_LL, _BB), lambda il, ib: (il, ib)),
          pl.BlockSpec((_LL, _LD, 1), lambda il, ib: (il, 0, 0)),
      ],
      out_specs=pl.BlockSpec((_LL, _LD, _BB), lambda il, ib: (il, 0, ib)),
      out_shape=jax.ShapeDtypeStruct((l, _LD, b), jnp.float32),
  )(g3, ids_t, pos3)


def kernel(token_ids, embed_table, proj_w, proj_b, pos_embed):
  b, l = token_ids.shape
  n = b * l
  assert n % (_NW * _CHUNK) == 0
  nch = n // (_NW * _CHUNK)
  ids = token_ids.astype(jnp.int32)
  table_t = embed_table.T
  z = jnp.zeros((_ED, _ED), jnp.float32)
  wd = jnp.concatenate([jnp.concatenate([proj_w, z], 1),
                        jnp.concatenate([z, proj_w], 1)], 0)
  p2 = _project(table_t, wd)
  phys = jnp.where(ids < _D, ids, ids - _D)
  ids3 = phys.reshape(_NW, nch, _CHUNK)
  g = _sc_gather(p2, ids3)
  g3 = g.reshape(b, l, _GD)
  ids_t = ids.T
  pos_bt = jnp.transpose(pos_embed, (0, 2, 1))[0]
  pos3 = (pos_bt[:, :l].T + proj_b[None, :])[:, :, None]
  out_t = _finish(g3, ids_t, pos3, b, l)
  return jnp.transpose(out_t, (2, 0, 1))


# speedup vs baseline: 2.4751x; 1.1499x over previous
import functools

import jax
import jax.numpy as jnp
from jax import lax
from jax.experimental import pallas as pl
from jax.experimental.pallas import tpu as pltpu
from jax.experimental.pallas import tpu_sc as plsc

_ED = 64
_LD = 64
_GD = 2 * _ED
_S = 16384
_NB = 31
_D = _NB * _S
_V = 1000000
_CHUNK = 128
_NC = 2
_NS = 16
_NW = _NC * _NS


def _project_body(alo_ref, ahi_ref, wd_ref, o_ref):
  i = pl.program_id(0)
  lane_g = lax.broadcasted_iota(jnp.int32, (_ED, _S), 1) + i * _S
  ahi = jnp.where(lane_g >= _V - _D, 0.0, ahi_ref[...])
  a2 = jnp.concatenate([alo_ref[...], ahi], axis=0)
  dn = (((0,), (0,)), ((), ()))
  o_ref[...] = lax.dot_general(a2, wd_ref[...], dn,
                               preferred_element_type=jnp.float32)


def _project(table_t, wd):
  return pl.pallas_call(
      _project_body,
      grid=(_NB,),
      in_specs=[
          pl.BlockSpec((_ED, _S), lambda i: (0, i)),
          pl.BlockSpec((_ED, _S), lambda i: (0, jnp.minimum(i + _NB, _V // _S))),
          pl.BlockSpec((_GD, _GD), lambda i: (0, 0)),
      ],
      out_specs=pl.BlockSpec((_S, _GD), lambda i: (i, 0)),
      out_shape=jax.ShapeDtypeStruct((_D, _GD), jnp.float32),
  )(table_t, table_t, wd)


def _gather_body(nch, table_hbm, idx_hbm, out_hbm, idx_v, rows_v, sem0, sem1):
  wid = lax.axis_index("s") * _NC + lax.axis_index("c")
  base = wid * (nch * _CHUNK)
  pltpu.sync_copy(idx_hbm.at[wid], idx_v)

  sems = (sem0, sem1)

  def start(c, buf):
    pltpu.make_async_copy(
        table_hbm.at[idx_v.at[c]], rows_v.at[buf], sems[buf]).start()

  def wait(c, buf):
    pltpu.make_async_copy(
        table_hbm.at[idx_v.at[c]], rows_v.at[buf], sems[buf]).wait()

  def store(c, buf):
    pltpu.sync_copy(rows_v.at[buf], out_hbm.at[pl.ds(base + c * _CHUNK, _CHUNK)])

  start(0, 0)

  def body(g, carry):
    c = 2 * g
    start(c + 1, 1)
    wait(c, 0)
    store(c, 0)

    @pl.when(c + 2 < nch)
    def _():
      start(c + 2, 0)

    wait(c + 1, 1)
    store(c + 1, 1)
    return carry

  lax.fori_loop(0, nch // 2, body, 0)


def _sc_gather(p2, ids3):
  nw, nch, chunk = ids3.shape
  n = nw * nch * chunk
  mesh = plsc.VectorSubcoreMesh(core_axis_name="c", subcore_axis_name="s")
  f = functools.partial(
      pl.kernel,
      mesh=mesh,
      out_type=jax.ShapeDtypeStruct((n, _GD), jnp.float32),
      scratch_types=[
          pltpu.VMEM((nch, chunk), jnp.int32),
          pltpu.VMEM((2, chunk, _GD), jnp.float32),
          pltpu.SemaphoreType.DMA,
          pltpu.SemaphoreType.DMA,
      ],
  )(functools.partial(_gather_body, nch))
  return f(p2, ids3)


_BB = 128
_LL = 40


def _finish_body(g_ref, ids_ref, pos_ref, o_ref):
  eye = (lax.broadcasted_iota(jnp.int32, (_BB, _BB), 0) ==
         lax.broadcasted_iota(jnp.int32, (_BB, _BB), 1)).astype(jnp.float32)
  dn = (((0,), (0,)), ((), ()))
  yts = []
  for li in range(_LL):
    gl = g_ref[:, li, :]
    yts.append(lax.dot_general(gl, eye, dn, preferred_element_type=jnp.float32))
  yt = jnp.stack(yts, axis=0)
  par = (ids_ref[...] >= _D)[:, None, :]
  j = lax.broadcasted_iota(jnp.int32, (_LL, _GD, _BB), 1)
  ym = jnp.where((j >= _ED) == par, yt, 0.0)
  yv = ym[:, :_ED, :] + ym[:, _ED:, :]
  o_ref[...] = yv + pos_ref[...]


def _finish(g3, ids_t, pos3, b, l):
  grid = (l // _LL, b // _BB)
  return pl.pallas_call(
      _finish_body,
      grid=grid,
      in_specs=[
          pl.BlockSpec((_BB, _LL, _GD), lambda il, ib: (ib, il, 0)),
          pl.BlockSpec((_LL, _BB), lambda il, ib: (il, ib)),
          pl.BlockSpec((_LL, _LD, 1), lambda il, ib: (il, 0, 0)),
      ],
      out_specs=pl.BlockSpec((_LL, _LD, _BB), lambda il, ib: (il, 0, ib)),
      out_shape=jax.ShapeDtypeStruct((l, _LD, b), jnp.float32),
  )(g3, ids_t, pos3)


def kernel(token_ids, embed_table, proj_w, proj_b, pos_embed):
  b, l = token_ids.shape
  n = b * l
  assert n % (_NW * _CHUNK) == 0
  nch = n // (_NW * _CHUNK)
  ids = token_ids.astype(jnp.int32)
  table_t = embed_table.T
  z = jnp.zeros((_ED, _ED), jnp.float32)
  wd = jnp.concatenate([jnp.concatenate([proj_w, z], 1),
                        jnp.concatenate([z, proj_w], 1)], 0)
  p2 = _project(table_t, wd)
  phys = jnp.where(ids < _D, ids, ids - _D)
  ids3 = phys.reshape(_NW, nch, _CHUNK)
  g = _sc_gather(p2, ids3)
  g3 = g.reshape(b, l, _GD)
  ids_t = ids.T
  pos_bt = jnp.transpose(pos_embed, (0, 2, 1))[0]
  pos3 = (pos_bt[:, :l].T + proj_b[None, :])[:, :, None]
  out_t = _finish(g3, ids_t, pos3, b, l)
  return jnp.transpose(out_t, (2, 0, 1))


# speedup vs baseline: 2.5509x; 1.0306x over previous
import functools

import jax
import jax.numpy as jnp
from jax import lax
from jax.experimental import pallas as pl
from jax.experimental.pallas import tpu as pltpu
from jax.experimental.pallas import tpu_sc as plsc

_ED = 64
_LD = 64
_GD = 2 * _ED
_S = 16384
_NB = 31
_D = _NB * _S
_V = 1000000
_CHUNK = 128
_NC = 2
_NS = 16
_NW = _NC * _NS


def _project_body(alo_ref, ahi_ref, wd_ref, o_ref):
  i = pl.program_id(0)
  lane_g = lax.broadcasted_iota(jnp.int32, (_ED, _S), 1) + i * _S
  ahi = jnp.where(lane_g >= _V - _D, 0.0, ahi_ref[...])
  a2 = jnp.concatenate([alo_ref[...], ahi], axis=0)
  dn = (((0,), (0,)), ((), ()))
  o_ref[...] = lax.dot_general(a2, wd_ref[...], dn,
                               preferred_element_type=jnp.float32)


def _project(table_t, wd):
  return pl.pallas_call(
      _project_body,
      grid=(_NB,),
      in_specs=[
          pl.BlockSpec((_ED, _S), lambda i: (0, i)),
          pl.BlockSpec((_ED, _S), lambda i: (0, jnp.minimum(i + _NB, _V // _S))),
          pl.BlockSpec((_GD, _GD), lambda i: (0, 0)),
      ],
      out_specs=pl.BlockSpec((_S, _GD), lambda i: (i, 0)),
      out_shape=jax.ShapeDtypeStruct((_D, _GD), jnp.float32),
  )(table_t, table_t, wd)


def _gather_body(nch, table_hbm, idx_hbm, out_hbm, idx_v, rows_v, sem0, sem1):
  wid = lax.axis_index("s") * _NC + lax.axis_index("c")
  base = wid * (nch * _CHUNK)
  pltpu.sync_copy(idx_hbm.at[wid], idx_v)

  sems = (sem0, sem1)

  def start(c, buf):
    pltpu.make_async_copy(
        table_hbm.at[idx_v.at[c]], rows_v.at[buf], sems[buf]).start()

  def wait(c, buf):
    pltpu.make_async_copy(
        table_hbm.at[idx_v.at[c]], rows_v.at[buf], sems[buf]).wait()

  def store(c, buf):
    pltpu.sync_copy(rows_v.at[buf], out_hbm.at[pl.ds(base + c * _CHUNK, _CHUNK)])

  start(0, 0)

  def body(g, carry):
    c = 2 * g
    start(c + 1, 1)
    wait(c, 0)
    store(c, 0)

    @pl.when(c + 2 < nch)
    def _():
      start(c + 2, 0)

    wait(c + 1, 1)
    store(c + 1, 1)
    return carry

  lax.fori_loop(0, nch // 2, body, 0)
  if nch % 2 == 1:
    wait(nch - 1, 0)
    store(nch - 1, 0)


def _sc_gather(p2, ids3):
  nw, nch, chunk = ids3.shape
  n = nw * nch * chunk
  mesh = plsc.VectorSubcoreMesh(core_axis_name="c", subcore_axis_name="s")
  f = functools.partial(
      pl.kernel,
      mesh=mesh,
      out_type=jax.ShapeDtypeStruct((n, _GD), jnp.float32),
      scratch_types=[
          pltpu.VMEM((nch, chunk), jnp.int32),
          pltpu.VMEM((2, chunk, _GD), jnp.float32),
          pltpu.SemaphoreType.DMA,
          pltpu.SemaphoreType.DMA,
      ],
  )(functools.partial(_gather_body, nch))
  return f(p2, ids3)


_BB = 128
_LL = 40


def _finish_body(g_ref, ids_ref, pos_ref, *rest):
  o_ref = rest[-1]
  eye = (lax.broadcasted_iota(jnp.int32, (_BB, _BB), 0) ==
         lax.broadcasted_iota(jnp.int32, (_BB, _BB), 1)).astype(jnp.float32)
  dn = (((0,), (0,)), ((), ()))
  yts = []
  for li in range(_LL):
    gl = g_ref[:, li, :]
    yts.append(lax.dot_general(gl, eye, dn, preferred_element_type=jnp.float32))
  yt = jnp.stack(yts, axis=0)
  par = (ids_ref[...] >= _D)[:, None, :]
  j = lax.broadcasted_iota(jnp.int32, (_LL, _GD, _BB), 1)
  ym = jnp.where((j >= _ED) == par, yt, 0.0)
  yv = ym[:, :_ED, :] + ym[:, _ED:, :]
  o_ref[...] = yv + pos_ref[...]


def _finish(g3, ids_t, pos3, b, l, b_off, alias_in=None):
  bs = g3.shape[0]
  grid = (l // _LL, bs // _BB)
  ins = [g3, ids_t, pos3]
  in_specs = [
      pl.BlockSpec((_BB, _LL, _GD), lambda il, ib: (ib, il, 0)),
      pl.BlockSpec((_LL, _BB), lambda il, ib: (il, ib + b_off)),
      pl.BlockSpec((_LL, _LD, 1), lambda il, ib: (il, 0, 0)),
  ]
  kwargs = {}
  if alias_in is not None:
    ins.append(alias_in)
    in_specs.append(pl.BlockSpec(memory_space=pl.ANY))
    kwargs["input_output_aliases"] = {3: 0}
  return pl.pallas_call(
      _finish_body,
      grid=grid,
      in_specs=in_specs,
      out_specs=pl.BlockSpec((_LL, _LD, _BB), lambda il, ib: (il, 0, ib + b_off)),
      out_shape=jax.ShapeDtypeStruct((l, _LD, b), jnp.float32),
      **kwargs,
  )(*ins)


def kernel(token_ids, embed_table, proj_w, proj_b, pos_embed):
  b, l = token_ids.shape
  n = b * l
  assert n % (_NW * _CHUNK) == 0
  nch = n // (_NW * _CHUNK)
  ids = token_ids.astype(jnp.int32)
  table_t = embed_table.T
  z = jnp.zeros((_ED, _ED), jnp.float32)
  wd = jnp.concatenate([jnp.concatenate([proj_w, z], 1),
                        jnp.concatenate([z, proj_w], 1)], 0)
  p2 = _project(table_t, wd)
  phys = jnp.where(ids < _D, ids, ids - _D)
  ids_t = ids.T
  pos_bt = jnp.transpose(pos_embed, (0, 2, 1))[0]
  pos3 = (pos_bt[:, :l].T + proj_b[None, :])[:, :, None]
  bh = b // 2
  out_t = None
  for h in range(2):
    ph = phys[h * bh:(h + 1) * bh].reshape(_NW, nch // 2, _CHUNK)
    g3 = _sc_gather(p2, ph).reshape(bh, l, _GD)
    out_t = _finish(g3, ids_t, pos3, b, l, h * (bh // _BB), alias_in=out_t)
  return jnp.transpose(out_t, (2, 0, 1))
